# Initial kernel scaffold; baseline (speedup 1.0000x reference)
import jax
import jax.numpy as jnp
from jax.experimental import pallas as pl


def kernel(x, W_qkv, b_qkv, W_proj, b_proj):
    raise NotImplementedError("write your pallas kernel here")


# speedup vs baseline: 1.4881x; 1.4881x over previous
import jax
import jax.numpy as jnp
from jax import lax
from jax.experimental import pallas as pl
from jax.experimental.pallas import tpu as pltpu

DIM = 96
NUM_HEADS = 8
HEAD_DIM = DIM // NUM_HEADS
WS = 8
TOPK = 4
H = W = 224
GH = GW = H // WS
R = GH * GW
T = WS * WS
NPIX = H * W
SCALE = HEAD_DIM ** (-0.5)

_PIX_BLK = 3584
_RB = 8


def _qkv_body(x_ref, w_ref, b_ref, o_ref):
    o_ref[...] = lax.dot_general(
        w_ref[...], x_ref[...], (((1,), (0,)), ((), ())),
        preferred_element_type=jnp.float32) + b_ref[...]


def _routing_body(q_ref, k_ref, idx_ref):
    q_r = jnp.mean(q_ref[0], axis=2)
    k_r = jnp.mean(k_ref[0], axis=2)
    s = lax.dot_general(q_r, k_r, (((1,), (1,)), ((), ())),
                        preferred_element_type=jnp.float32) * SCALE
    iota = lax.broadcasted_iota(jnp.int32, (R, R), 1)
    cols = []
    for _ in range(TOPK):
        m = jnp.max(s, axis=1, keepdims=True)
        idx = jnp.min(jnp.where(s == m, iota, R), axis=1)
        cols.append(idx)
        s = jnp.where(iota == idx[:, None], -jnp.inf, s)
    idx_ref[0] = jnp.stack(cols, axis=1).astype(jnp.int32)


def _attn_body(idx_ref, q_ref, k_ref, v_ref, o_ref):
    for i in range(_RB):
        q = q_ref[0, i]
        ks = jnp.concatenate(
            [k_ref[0, idx_ref[0, 0, 0, TOPK * i + j]] for j in range(TOPK)],
            axis=1)
        vs = jnp.concatenate(
            [v_ref[0, idx_ref[0, 0, 0, TOPK * i + j]] for j in range(TOPK)],
            axis=1)
        s = lax.dot_general(q, ks, (((0,), (0,)), ((), ())),
                            preferred_element_type=jnp.float32) * SCALE
        m = jnp.max(s, axis=1, keepdims=True)
        p = jnp.exp(s - m)
        p = p / jnp.sum(p, axis=1, keepdims=True)
        o_ref[0, i] = lax.dot_general(vs, p, (((1,), (1,)), ((), ())),
                                      preferred_element_type=jnp.float32)


def kernel(x, W_qkv, b_qkv, W_proj, b_proj):
    x2 = x.reshape(DIM, NPIX)

    qkv = pl.pallas_call(
        _qkv_body,
        grid=(NPIX // _PIX_BLK,),
        in_specs=[
            pl.BlockSpec((DIM, _PIX_BLK), lambda j: (0, j)),
            pl.BlockSpec((3 * DIM, DIM), lambda j: (0, 0)),
            pl.BlockSpec((3 * DIM, 1), lambda j: (0, 0)),
        ],
        out_specs=pl.BlockSpec((3 * DIM, _PIX_BLK), lambda j: (0, j)),
        out_shape=jax.ShapeDtypeStruct((3 * DIM, NPIX), jnp.float32),
    )(x2, W_qkv, b_qkv.reshape(3 * DIM, 1))

    qkv_w = qkv.reshape(3, NUM_HEADS, HEAD_DIM, GH, WS, GW, WS)
    qkv_w = jnp.transpose(qkv_w, (0, 1, 3, 5, 2, 4, 6))
    qkv_w = qkv_w.reshape(3, NUM_HEADS, R, HEAD_DIM, T)
    q_w, k_w, v_w = qkv_w[0], qkv_w[1], qkv_w[2]

    topk_idx = pl.pallas_call(
        _routing_body,
        grid=(NUM_HEADS,),
        in_specs=[
            pl.BlockSpec((1, R, HEAD_DIM, T), lambda h: (h, 0, 0, 0)),
            pl.BlockSpec((1, R, HEAD_DIM, T), lambda h: (h, 0, 0, 0)),
        ],
        out_specs=pl.BlockSpec((1, R, TOPK), lambda h: (h, 0, 0)),
        out_shape=jax.ShapeDtypeStruct((NUM_HEADS, R, TOPK), jnp.int32),
    )(q_w, k_w)

    idx_smem = topk_idx.reshape(NUM_HEADS, R // _RB, 1, _RB * TOPK)

    out_w = pl.pallas_call(
        _attn_body,
        grid=(NUM_HEADS, R // _RB),
        in_specs=[
            pl.BlockSpec((1, 1, 1, _RB * TOPK), lambda h, rb: (h, rb, 0, 0),
                         memory_space=pltpu.SMEM),
            pl.BlockSpec((1, _RB, HEAD_DIM, T), lambda h, rb: (h, rb, 0, 0)),
            pl.BlockSpec((1, R, HEAD_DIM, T), lambda h, rb: (h, 0, 0, 0)),
            pl.BlockSpec((1, R, HEAD_DIM, T), lambda h, rb: (h, 0, 0, 0)),
        ],
        out_specs=pl.BlockSpec((1, _RB, HEAD_DIM, T),
                               lambda h, rb: (h, rb, 0, 0)),
        out_shape=jax.ShapeDtypeStruct((NUM_HEADS, R, HEAD_DIM, T),
                                       jnp.float32),
    )(idx_smem, q_w, k_w, v_w)

    out_m = out_w.reshape(NUM_HEADS, GH, GW, HEAD_DIM, WS, WS)
    out_m = jnp.transpose(out_m, (0, 3, 1, 4, 2, 5)).reshape(DIM, NPIX)

    out = pl.pallas_call(
        _qkv_body,
        grid=(NPIX // _PIX_BLK,),
        in_specs=[
            pl.BlockSpec((DIM, _PIX_BLK), lambda j: (0, j)),
            pl.BlockSpec((DIM, DIM), lambda j: (0, 0)),
            pl.BlockSpec((DIM, 1), lambda j: (0, 0)),
        ],
        out_specs=pl.BlockSpec((DIM, _PIX_BLK), lambda j: (0, j)),
        out_shape=jax.ShapeDtypeStruct((DIM, NPIX), jnp.float32),
    )(out_m, W_proj, b_proj.reshape(DIM, 1))

    return out.reshape(1, DIM, H, W)


# speedup vs baseline: 6.1917x; 4.1608x over previous
import jax
import jax.numpy as jnp
from jax import lax
from jax.experimental import pallas as pl
from jax.experimental.pallas import tpu as pltpu

DIM = 96
NUM_HEADS = 8
HEAD_DIM = DIM // NUM_HEADS
WS = 8
TOPK = 4
H = W = 224
GH = GW = H // WS
R = GH * GW
T = WS * WS
NPIX = H * W
SCALE = HEAD_DIM ** (-0.5)

_GHB = 2
_RBLK = _GHB * GW
_CBLK = _GHB * WS * W
_RB = 8


def _qkv_body(x_ref, w_ref, b_ref, o_ref):
    mm = lax.dot_general(w_ref[...], x_ref[...], (((1,), (0,)), ((), ())),
                         preferred_element_type=jnp.float32) + b_ref[...]
    for g2 in range(_GHB):
        for gwi in range(GW):
            base = g2 * WS * W + gwi * WS
            cols = jnp.concatenate(
                [lax.slice(mm, (0, base + ty * W), (3 * DIM, base + ty * W + WS))
                 for ty in range(WS)], axis=1)
            o_ref[:, :, g2 * GW + gwi] = cols.reshape(
                3, NUM_HEADS, HEAD_DIM, T)


def _routing_body(q_ref, k_ref, idx_ref):
    q_r = jnp.mean(q_ref[0, 0], axis=2)
    k_r = jnp.mean(k_ref[0, 0], axis=2)
    s = lax.dot_general(q_r, k_r, (((1,), (1,)), ((), ())),
                        preferred_element_type=jnp.float32) * SCALE
    iota = lax.broadcasted_iota(jnp.int32, (R, R), 1)
    cols = []
    for _ in range(TOPK):
        m = jnp.max(s, axis=1, keepdims=True)
        idx = jnp.min(jnp.where(s == m, iota, R), axis=1)
        cols.append(idx)
        s = jnp.where(iota == idx[:, None], -jnp.inf, s)
    idx_ref[0] = jnp.stack(cols, axis=1).astype(jnp.int32)


def _attn_body(idx_ref, q_ref, k_ref, v_ref, o_ref):
    for i in range(_RB):
        q = q_ref[0, 0, i]
        ks = jnp.concatenate(
            [k_ref[0, 0, idx_ref[0, 0, 0, TOPK * i + j]] for j in range(TOPK)],
            axis=1)
        vs = jnp.concatenate(
            [v_ref[0, 0, idx_ref[0, 0, 0, TOPK * i + j]] for j in range(TOPK)],
            axis=1)
        s = lax.dot_general(q, ks, (((0,), (0,)), ((), ())),
                            preferred_element_type=jnp.float32) * SCALE
        m = jnp.max(s, axis=1, keepdims=True)
        p = jnp.exp(s - m)
        p = p / jnp.sum(p, axis=1, keepdims=True)
        o_ref[0, i] = lax.dot_general(vs, p, (((1,), (1,)), ((), ())),
                                      preferred_element_type=jnp.float32)


def _proj_body(o_ref, w_ref, b_ref, out_ref):
    chunks = [o_ref[:, r].reshape(DIM, T) for r in range(_RBLK)]
    ow = jnp.concatenate(
        [jnp.concatenate(
            [chunks[(y // WS) * GW + g][:, (y % WS) * WS:(y % WS + 1) * WS]
             for g in range(GW)], axis=1)
         for y in range(_GHB * WS)], axis=1)
    out_ref[...] = lax.dot_general(
        w_ref[...], ow, (((1,), (0,)), ((), ())),
        preferred_element_type=jnp.float32) + b_ref[...]


def kernel(x, W_qkv, b_qkv, W_proj, b_proj):
    x2 = x.reshape(DIM, NPIX)

    qkv_w = pl.pallas_call(
        _qkv_body,
        grid=(GH // _GHB,),
        in_specs=[
            pl.BlockSpec((DIM, _CBLK), lambda j: (0, j)),
            pl.BlockSpec((3 * DIM, DIM), lambda j: (0, 0)),
            pl.BlockSpec((3 * DIM, 1), lambda j: (0, 0)),
        ],
        out_specs=pl.BlockSpec((3, NUM_HEADS, _RBLK, HEAD_DIM, T),
                               lambda j: (0, 0, j, 0, 0)),
        out_shape=jax.ShapeDtypeStruct((3, NUM_HEADS, R, HEAD_DIM, T),
                                       jnp.float32),
    )(x2, W_qkv, b_qkv.reshape(3 * DIM, 1))

    topk_idx = pl.pallas_call(
        _routing_body,
        grid=(NUM_HEADS,),
        in_specs=[
            pl.BlockSpec((1, 1, R, HEAD_DIM, T), lambda h: (0, h, 0, 0, 0)),
            pl.BlockSpec((1, 1, R, HEAD_DIM, T), lambda h: (1, h, 0, 0, 0)),
        ],
        out_specs=pl.BlockSpec((1, R, TOPK), lambda h: (h, 0, 0)),
        out_shape=jax.ShapeDtypeStruct((NUM_HEADS, R, TOPK), jnp.int32),
    )(qkv_w, qkv_w)

    idx_smem = topk_idx.reshape(NUM_HEADS, R // _RB, 1, _RB * TOPK)

    out_w = pl.pallas_call(
        _attn_body,
        grid=(NUM_HEADS, R // _RB),
        in_specs=[
            pl.BlockSpec((1, 1, 1, _RB * TOPK), lambda h, rb: (h, rb, 0, 0),
                         memory_space=pltpu.SMEM),
            pl.BlockSpec((1, 1, _RB, HEAD_DIM, T),
                         lambda h, rb: (0, h, rb, 0, 0)),
            pl.BlockSpec((1, 1, R, HEAD_DIM, T),
                         lambda h, rb: (1, h, 0, 0, 0)),
            pl.BlockSpec((1, 1, R, HEAD_DIM, T),
                         lambda h, rb: (2, h, 0, 0, 0)),
        ],
        out_specs=pl.BlockSpec((1, _RB, HEAD_DIM, T),
                               lambda h, rb: (h, rb, 0, 0)),
        out_shape=jax.ShapeDtypeStruct((NUM_HEADS, R, HEAD_DIM, T),
                                       jnp.float32),
    )(idx_smem, qkv_w, qkv_w, qkv_w)

    out = pl.pallas_call(
        _proj_body,
        grid=(GH // _GHB,),
        in_specs=[
            pl.BlockSpec((NUM_HEADS, _RBLK, HEAD_DIM, T),
                         lambda j: (0, j, 0, 0)),
            pl.BlockSpec((DIM, DIM), lambda j: (0, 0)),
            pl.BlockSpec((DIM, 1), lambda j: (0, 0)),
        ],
        out_specs=pl.BlockSpec((DIM, _CBLK), lambda j: (0, j)),
        out_shape=jax.ShapeDtypeStruct((DIM, NPIX), jnp.float32),
    )(out_w, W_proj, b_proj.reshape(DIM, 1))

    return out.reshape(1, DIM, H, W)
